# baseline (speedup 1.0000x reference)
import functools

import jax
import jax.numpy as jnp
from jax import lax
from jax.experimental import pallas as pl
from jax.experimental.pallas import tpu as pltpu
from jax.experimental.pallas import tpu_sc as plsc

MAXLEN = 200
EMBED_DIM = 128
BATCH = 1024
VOCAB_PARENT = 1000
NC = 2
NS = 16
NW = NC * NS
CHUNK = 40
SPLITS = MAXLEN // CHUNK
SEQ_PER_W = BATCH // NW
CPW = SEQ_PER_W * SPLITS
NBUF = SPLITS
NIT = CPW // NBUF
LANES = 16
EMBED_W = EMBED_DIM // 2
HI_MASK = jnp.int32(-65536)


def _pack_bf16_pairs(t):
    v, d = t.shape
    t = t.astype(jnp.bfloat16).reshape(v, d // 32, 2, 16)
    t = t.transpose(0, 1, 3, 2)
    return lax.bitcast_convert_type(t, jnp.int32).reshape(v, d // 2)


def _expand_halves(w):
    lo = lax.bitcast_convert_type(lax.shift_left(w, 16), jnp.float32)
    hi = lax.bitcast_convert_type(lax.bitwise_and(w, HI_MASK), jnp.float32)
    return lo, hi


def kernel(x, y, token_table, pos_table, parent_table):
    mesh = plsc.VectorSubcoreMesh(core_axis_name="c", subcore_axis_name="s")

    @functools.partial(
        pl.kernel,
        out_type=jax.ShapeDtypeStruct((BATCH, MAXLEN, EMBED_DIM), jnp.float32),
        mesh=mesh,
        compiler_params=pltpu.CompilerParams(use_tc_tiling_on_sc=False,
                                             needs_layout_passes=False),
        scratch_types=[
            pltpu.VMEM((CPW, CHUNK), jnp.int32),
            pltpu.VMEM((CPW, CHUNK), jnp.int32),
            [pltpu.VMEM((CHUNK, EMBED_DIM), jnp.float32) for _ in range(NBUF)],
            pltpu.VMEM((VOCAB_PARENT, EMBED_W), jnp.int32),
            pltpu.VMEM((MAXLEN, EMBED_W), jnp.int32),
            [pltpu.SemaphoreType.DMA for _ in range(NBUF)],
            [pltpu.SemaphoreType.DMA for _ in range(NBUF)],
        ],
    )
    def k(x_hbm, y_hbm, tok_hbm, pos_hbm, par_hbm, out_hbm,
          idx_x, idx_y, tok_v, par_t, pos_v, sem_g, sem_o):
        wid = lax.axis_index("s") * NC + lax.axis_index("c")
        seq0 = wid * SEQ_PER_W
        chunk0 = wid * CPW
        pltpu.sync_copy(x_hbm.at[pl.ds(chunk0, CPW)], idx_x)
        pltpu.sync_copy(y_hbm.at[pl.ds(chunk0, CPW)], idx_y)
        pltpu.sync_copy(pos_hbm, pos_v)
        pltpu.sync_copy(par_hbm, par_t)

        def issue(it, p, q):
            c = it * NBUF + p
            pltpu.async_copy(tok_hbm.at[idx_x.at[c]], tok_v[q], sem_g[q])

        def wait_gather(q):
            pltpu.make_async_copy(
                tok_hbm.at[idx_x.at[0]], tok_v[q], sem_g[q]).wait()

        def wait_out(q):
            pltpu.make_async_copy(
                tok_v[q], out_hbm.at[0, pl.ds(0, CHUNK)], sem_o[q]).wait()

        issue(0, 0, 0)
        issue(0, 1, 1)

        cols = [lax.iota(jnp.int32, 16) + g * LANES
                for g in range(EMBED_W // LANES)]

        @pl.loop(0, NIT)
        def _it(it):
            for p in range(NBUF):
                q = (p + 2) % NBUF
                it2 = it + (p + 2) // NBUF
                if p < NBUF - 2:
                    @pl.when(it > 0)
                    def _():
                        wait_out(q)
                    issue(it2, q, q)
                else:
                    @pl.when(it < NIT - 1)
                    def _():
                        wait_out(q)
                        issue(it2, q, q)

                wait_gather(p)
                c = it * NBUF + p

                @pl.loop(0, CHUNK, unroll=2)
                def _row(r):
                    w0 = jnp.minimum((r // LANES) * LANES, CHUNK - LANES)
                    yv = idx_y[c, pl.ds(w0, LANES)]
                    lane = jnp.broadcast_to(r - w0, (LANES,)).astype(jnp.int32)
                    yr = yv.at[lane].get(mode="promise_in_bounds")
                    for g in range(EMBED_W // LANES):
                        yw = plsc.load_gather(par_t, [yr, cols[g]])
                        ow = pos_v[p * CHUNK + r, pl.ds(g * LANES, LANES)]
                        ya, yb = _expand_halves(yw)
                        oa, ob = _expand_halves(ow)
                        sa = pl.ds(g * 32, LANES)
                        sb = pl.ds(g * 32 + LANES, LANES)
                        plsc.addupdate(tok_v[p].at[r, sa], ya + oa)
                        plsc.addupdate(tok_v[p].at[r, sb], yb + ob)

                pltpu.async_copy(
                    tok_v[p],
                    out_hbm.at[seq0 + it, pl.ds(p * CHUNK, CHUNK)],
                    sem_o[p])

        for p in range(NBUF):
            wait_out(p)

    x2 = x.reshape(BATCH * SPLITS, CHUNK)
    y2 = y.reshape(BATCH * SPLITS, CHUNK)
    return k(x2, y2, token_table,
             _pack_bf16_pairs(pos_table), _pack_bf16_pairs(parent_table))


# speedup vs baseline: 1.4905x; 1.4905x over previous
import functools

import jax
import jax.numpy as jnp
from jax import lax
from jax.experimental import pallas as pl
from jax.experimental.pallas import tpu as pltpu
from jax.experimental.pallas import tpu_sc as plsc

MAXLEN = 200
EMBED_DIM = 128
BATCH = 1024
NC = 2
NS = 16
NW = NC * NS
CHUNK = 40
SPLITS = MAXLEN // CHUNK
SEQ_PER_W = BATCH // NW
CPW = SEQ_PER_W * SPLITS
NBUF = SPLITS
NIT = CPW // NBUF
LANES = 16


def kernel(x, y, token_table, pos_table, parent_table):
    mesh = plsc.VectorSubcoreMesh(core_axis_name="c", subcore_axis_name="s")

    @functools.partial(
        pl.kernel,
        out_type=jax.ShapeDtypeStruct((BATCH, MAXLEN, EMBED_DIM), jnp.float32),
        mesh=mesh,
        compiler_params=pltpu.CompilerParams(use_tc_tiling_on_sc=False),
        scratch_types=[
            pltpu.VMEM((CPW, CHUNK), jnp.int32),
            pltpu.VMEM((CPW, CHUNK), jnp.int32),
            [pltpu.VMEM((CHUNK, EMBED_DIM), jnp.float32) for _ in range(NBUF)],
            [pltpu.VMEM((CHUNK, EMBED_DIM), jnp.float32) for _ in range(NBUF)],
            pltpu.VMEM((MAXLEN, EMBED_DIM), jnp.float32),
            [pltpu.SemaphoreType.DMA for _ in range(NBUF)],
            [pltpu.SemaphoreType.DMA for _ in range(NBUF)],
        ],
    )
    def k(x_hbm, y_hbm, tok_hbm, pos_hbm, par_hbm, out_hbm,
          idx_x, idx_y, tok_v, par_v, pos_v, sem_g, sem_o):
        wid = lax.axis_index("s") * NC + lax.axis_index("c")
        seq0 = wid * SEQ_PER_W
        chunk0 = wid * CPW
        pltpu.sync_copy(x_hbm.at[pl.ds(chunk0, CPW)], idx_x)
        pltpu.sync_copy(y_hbm.at[pl.ds(chunk0, CPW)], idx_y)
        pltpu.sync_copy(pos_hbm, pos_v)

        def issue(it, p, q):
            c = it * NBUF + p
            pltpu.async_copy(tok_hbm.at[idx_x.at[c]], tok_v[q], sem_g[q])
            pltpu.async_copy(par_hbm.at[idx_y.at[c]], par_v[q], sem_g[q])

        def wait_gather(q):
            pltpu.make_async_copy(
                tok_hbm.at[idx_x.at[0]], tok_v[q], sem_g[q]).wait()
            pltpu.make_async_copy(
                par_hbm.at[idx_y.at[0]], par_v[q], sem_g[q]).wait()

        def wait_out(q):
            pltpu.make_async_copy(
                tok_v[q], out_hbm.at[0, pl.ds(0, CHUNK)], sem_o[q]).wait()

        issue(0, 0, 0)
        issue(0, 1, 1)

        @pl.loop(0, NIT)
        def _it(it):
            for p in range(NBUF):
                q = (p + 2) % NBUF
                it2 = it + (p + 2) // NBUF
                if p < NBUF - 2:
                    @pl.when(it > 0)
                    def _():
                        wait_out(q)
                    issue(it2, q, q)
                else:
                    @pl.when(it < NIT - 1)
                    def _():
                        wait_out(q)
                        issue(it2, q, q)

                wait_gather(p)

                @pl.loop(0, CHUNK, unroll=2)
                def _row(r):
                    for cb in range(EMBED_DIM // LANES):
                        sl = pl.ds(cb * LANES, LANES)
                        plsc.addupdate(
                            tok_v[p].at[r, sl],
                            par_v[p][r, sl] + pos_v[p * CHUNK + r, sl])

                pltpu.async_copy(
                    tok_v[p],
                    out_hbm.at[seq0 + it, pl.ds(p * CHUNK, CHUNK)],
                    sem_o[p])

        for p in range(NBUF):
            wait_out(p)

    x2 = x.reshape(BATCH * SPLITS, CHUNK)
    y2 = y.reshape(BATCH * SPLITS, CHUNK)
    return k(x2, y2, token_table, pos_table, parent_table)
